# baseline (speedup 1.0000x reference)
import functools

import jax
import jax.numpy as jnp
from jax import lax
from jax.experimental import pallas as pl
from jax.experimental.pallas import tpu as pltpu
from jax.experimental.pallas import tpu_sc as plsc

EMB = 64
NUM_CORES = 2
NUM_SUBCORES = 16
NW = NUM_CORES * NUM_SUBCORES

G = 128
NGRP = G // 16


def _embedding_lookup(ids2d, table2):
    n_groups, _ = ids2d.shape
    n = n_groups * G
    grp_per_w = n_groups // NW

    mesh = plsc.VectorSubcoreMesh(
        core_axis_name="c", subcore_axis_name="s",
        num_cores=NUM_CORES, num_subcores=NUM_SUBCORES)

    @functools.partial(
        pl.kernel,
        mesh=mesh,
        out_type=jax.ShapeDtypeStruct((n, EMB), jnp.float32),
        scratch_types=[
            pltpu.VMEM((grp_per_w, G), jnp.int32),
            pltpu.VMEM((2, G), jnp.int32),
            pltpu.VMEM((2, G, 2 * EMB), jnp.float32),
            pltpu.VMEM((2, G, EMB), jnp.float32),
            pltpu.SemaphoreType.DMA((2,)),
            pltpu.SemaphoreType.DMA((2,)),
        ],
    )
    def emb_kernel(ids_hbm, tbl_hbm, out_hbm,
                   ids_v, hidx_v, slab_v, outb_v, gsem, wsem):
        wid = lax.axis_index("s") * NUM_CORES + lax.axis_index("c")
        base_grp = wid * grp_per_w
        pltpu.sync_copy(ids_hbm.at[pl.ds(base_grp, grp_per_w)], ids_v)

        def fire(g, b):
            for r in range(NGRP):
                ids16 = ids_v[g, pl.ds(16 * r, 16)]
                hidx_v[b, pl.ds(16 * r, 16)] = lax.shift_right_logical(
                    ids16, 1)
            pltpu.async_copy(tbl_hbm.at[hidx_v.at[b]], slab_v.at[b],
                             gsem.at[b])

        def drain(b):
            pltpu.make_async_copy(tbl_hbm.at[hidx_v.at[b]], slab_v.at[b],
                                  gsem.at[b]).wait()

        def wait_write(b):
            pltpu.make_async_copy(outb_v.at[b], out_hbm.at[pl.ds(0, G)],
                                  wsem.at[b]).wait()

        fire(0, 0)

        @pl.loop(0, grp_per_w, step=2)
        def _main(j):
            for b in range(2):
                g = j + b
                drain(b)

                @pl.when(g + 1 < grp_per_w)
                def _prefetch():
                    fire(g + 1, 1 - b)

                @pl.when(g >= 2)
                def _reclaim():
                    wait_write(b)

                slab = slab_v.at[b]
                outb = outb_v.at[b]

                @pl.loop(0, NGRP)
                def _groups(r):
                    base = r * 16
                    p16 = (ids_v[g, pl.ds(base, 16)] & 1).astype(
                        jnp.float32)
                    for i in range(16):
                        row = base + i
                        idx = jnp.full((16,), i, dtype=jnp.int32)
                        pv = p16.at[idx].get(mode="promise_in_bounds")
                        for s in range(EMB // 16):
                            lo = slab[row, pl.ds(16 * s, 16)]
                            hi = slab[row, pl.ds(EMB + 16 * s, 16)]
                            outb[row, pl.ds(16 * s, 16)] = (
                                lo + pv * (hi - lo))

                off = (base_grp + g) * G
                pltpu.async_copy(outb, out_hbm.at[pl.ds(off, G)],
                                 wsem.at[b])

        wait_write(0)
        wait_write(1)

    return emb_kernel(ids2d, table2)


def kernel(input_ids, table):
    b, s = input_ids.shape
    n = b * s
    ids2d = input_ids.astype(jnp.int32).reshape(n // G, G)
    table2 = lax.optimization_barrier(
        table.reshape(table.shape[0] // 2, 2 * EMB))
    out = _embedding_lookup(ids2d, table2)
    return out.reshape(b, s, EMB)


# speedup vs baseline: 1.0426x; 1.0426x over previous
import functools

import jax
import jax.numpy as jnp
from jax import lax
from jax.experimental import pallas as pl
from jax.experimental.pallas import tpu as pltpu
from jax.experimental.pallas import tpu_sc as plsc

EMB = 64
NUM_CORES = 2
NUM_SUBCORES = 16
NW = NUM_CORES * NUM_SUBCORES

G = 128
NGRP = G // 16


def _embedding_lookup(ids2d, table2):
    n_groups, _ = ids2d.shape
    n = n_groups * G
    grp_per_w = n_groups // NW

    mesh = plsc.VectorSubcoreMesh(
        core_axis_name="c", subcore_axis_name="s",
        num_cores=NUM_CORES, num_subcores=NUM_SUBCORES)

    @functools.partial(
        pl.kernel,
        mesh=mesh,
        out_type=jax.ShapeDtypeStruct((n, EMB), jnp.float32),
        scratch_types=[
            pltpu.VMEM((grp_per_w, G), jnp.int32),
            pltpu.VMEM((2, G), jnp.int32),
            pltpu.VMEM((2, G, 2 * EMB), jnp.float32),
            pltpu.VMEM((2, G, EMB), jnp.float32),
            pltpu.SemaphoreType.DMA((2,)),
            pltpu.SemaphoreType.DMA((2,)),
        ],
    )
    def emb_kernel(ids_hbm, tbl_hbm, out_hbm,
                   ids_v, hidx_v, slab_v, outb_v, gsem, wsem):
        wid = lax.axis_index("s") * NUM_CORES + lax.axis_index("c")
        base_grp = wid * grp_per_w
        pltpu.sync_copy(ids_hbm.at[pl.ds(base_grp, grp_per_w)], ids_v)

        def fire(g, b):
            for r in range(NGRP):
                ids16 = ids_v[g, pl.ds(16 * r, 16)]
                hidx_v[b, pl.ds(16 * r, 16)] = lax.shift_right_logical(
                    ids16, 1)
            pltpu.async_copy(tbl_hbm.at[hidx_v.at[b]], slab_v.at[b],
                             gsem.at[b])

        def drain(b):
            pltpu.make_async_copy(tbl_hbm.at[hidx_v.at[b]], slab_v.at[b],
                                  gsem.at[b]).wait()

        def wait_write(b):
            pltpu.make_async_copy(outb_v.at[b], out_hbm.at[pl.ds(0, G)],
                                  wsem.at[b]).wait()

        fire(0, 0)

        @pl.loop(0, grp_per_w, step=2)
        def _main(j):
            for b in range(2):
                g = j + b
                drain(b)

                @pl.when(g + 1 < grp_per_w)
                def _prefetch():
                    fire(g + 1, 1 - b)

                @pl.when(g >= 2)
                def _reclaim():
                    wait_write(b)

                slab = slab_v.at[b]
                outb = outb_v.at[b]

                @pl.loop(0, G)
                def _rows(row):
                    rg = lax.div(row, 16) * 16
                    lane = lax.rem(row, 16)
                    p16 = (ids_v[g, pl.ds(rg, 16)] & 1).astype(jnp.float32)
                    idx = jnp.full((16,), lane, dtype=jnp.int32)
                    pv = p16.at[idx].get(mode="promise_in_bounds")
                    for s in range(EMB // 16):
                        lo = slab[row, pl.ds(16 * s, 16)]
                        hi = slab[row, pl.ds(EMB + 16 * s, 16)]
                        outb[row, pl.ds(16 * s, 16)] = lo + pv * (hi - lo)

                off = (base_grp + g) * G
                pltpu.async_copy(outb, out_hbm.at[pl.ds(off, G)],
                                 wsem.at[b])

        wait_write(0)
        wait_write(1)

    return emb_kernel(ids2d, table2)


def kernel(input_ids, table):
    b, s = input_ids.shape
    n = b * s
    ids2d = input_ids.astype(jnp.int32).reshape(n // G, G)
    table2 = lax.optimization_barrier(
        table.reshape(table.shape[0] // 2, 2 * EMB))
    out = _embedding_lookup(ids2d, table2)
    return out.reshape(b, s, EMB)
